# baseline (device time: 105862 ns/iter reference)
import jax
import jax.numpy as jnp
from jax import lax
from jax.experimental import pallas as pl
from jax.experimental.pallas import tpu as pltpu

N_DEV = 16
M_BLK = 256
NSUB = 2
N_LANES = 2 * NSUB

RING = (0, 1, 5, 9, 13, 14, 10, 6, 2, 3, 7, 11, 15, 12, 8, 4)
POS = tuple(RING.index(l) for l in range(N_DEV))


def _lane_spans(n):
    head = n // 8
    return (
        (0, head, True),
        (n // 2, n // 2 + head, False),
        (head, n // 2, True),
        (n // 2 + head, n, False),
    )


def _lut(table, idx):
    out = jnp.int32(table[0])
    for q in range(1, N_DEV):
        out = jnp.where(idx == q, jnp.int32(table[q]), out)
    return out


class _Lane:

    def __init__(self, is_a, col0, col1, send, recv, send_sems, recv_sems):
        self.is_a = is_a
        self.col0 = col0
        self.col1 = col1
        self.send = send
        self.recv = recv
        self.send_sems = send_sems
        self.recv_sems = recv_sems
        self.rdmas = []


def kernel(x, w_mat):
    m, k = x.shape
    _, n = w_mat.shape

    def body(x_ref, w_ref, out_ref, x_bf, w_bf, *scratch):
        sends = scratch[:N_LANES]
        recvs = scratch[N_LANES:2 * N_LANES]
        send_sems = scratch[2 * N_LANES:3 * N_LANES]
        recv_sems = scratch[3 * N_LANES:4 * N_LANES]

        my = lax.axis_index("i")
        pos = _lut(POS, my)
        right = _lut(tuple(RING[(POS[l] + 1) % N_DEV] for l in range(N_DEV)),
                     my)
        left = _lut(tuple(RING[(POS[l] - 1) % N_DEV] for l in range(N_DEV)),
                    my)

        barrier = pltpu.get_barrier_semaphore()
        for nbr in (left, right):
            pl.semaphore_signal(
                barrier, inc=1, device_id=(nbr,),
                device_id_type=pl.DeviceIdType.MESH,
            )
        w_bf[...] = w_ref[...].astype(jnp.bfloat16)
        x_bf[...] = x_ref[...].astype(jnp.bfloat16)
        pl.semaphore_wait(barrier, 2)

        lanes = [
            _Lane(is_a, c0, c1, sends[i], recvs[i],
                  send_sems[i], recv_sems[i])
            for i, (c0, c1, is_a) in enumerate(_lane_spans(n))
        ]

        def gemm(c, lane):
            a = x_bf[pl.ds(c * M_BLK, M_BLK), :]
            return jnp.dot(a, w_bf[:, lane.col0:lane.col1],
                           preferred_element_type=jnp.float32)

        for t in range(N_DEV - 1):
            ca = _lut(tuple(RING[(q - 1 - t) % N_DEV] for q in range(N_DEV)),
                      pos)
            cb = _lut(tuple(RING[(q + 1 + t) % N_DEV] for q in range(N_DEV)),
                      pos)
            for lane in lanes:
                acc = gemm(ca if lane.is_a else cb, lane)
                if t > 0:
                    lane.rdmas[t - 1].wait_recv()
                    acc = acc + lane.recv[t - 1].astype(jnp.float32)
                if t >= 2:
                    lane.rdmas[t - 2].wait_send()
                lane.send[t % 2] = acc.astype(jnp.bfloat16)
                rdma = pltpu.make_async_remote_copy(
                    src_ref=lane.send.at[t % 2],
                    dst_ref=lane.recv.at[t],
                    send_sem=lane.send_sems.at[t % 2],
                    recv_sem=lane.recv_sems.at[t],
                    device_id=(right if lane.is_a else left,),
                    device_id_type=pl.DeviceIdType.MESH,
                )
                rdma.start()
                lane.rdmas.append(rdma)

        owns = [gemm(my, lane) for lane in lanes]
        for lane, own in zip(lanes, owns):
            lane.rdmas[N_DEV - 2].wait_recv()
            out_ref[:, lane.col0:lane.col1] = jnp.maximum(
                own + lane.recv[N_DEV - 2].astype(jnp.float32), 0.0)

        for lane in lanes:
            lane.rdmas[N_DEV - 3].wait_send()
            lane.rdmas[N_DEV - 2].wait_send()

    return pl.pallas_call(
        body,
        out_shape=jax.ShapeDtypeStruct((M_BLK, n), jnp.float32),
        in_specs=[
            pl.BlockSpec(memory_space=pltpu.VMEM),
            pl.BlockSpec(memory_space=pltpu.VMEM),
        ],
        out_specs=pl.BlockSpec(memory_space=pltpu.VMEM),
        scratch_shapes=(
            [pltpu.VMEM((m, k), jnp.bfloat16)]
            + [pltpu.VMEM((k, n), jnp.bfloat16)]
            + [pltpu.VMEM((2, M_BLK, c1 - c0), jnp.bfloat16)
               for c0, c1, _ in _lane_spans(n)]
            + [pltpu.VMEM((N_DEV - 1, M_BLK, c1 - c0), jnp.bfloat16)
               for c0, c1, _ in _lane_spans(n)]
            + [pltpu.SemaphoreType.DMA((2,))] * N_LANES
            + [pltpu.SemaphoreType.DMA((N_DEV - 1,))] * N_LANES
        ),
        compiler_params=pltpu.CompilerParams(collective_id=0),
    )(x, w_mat)


# device time: 95546 ns/iter; 1.1080x vs baseline; 1.1080x over previous
import jax
import jax.numpy as jnp
from jax import lax
from jax.experimental import pallas as pl
from jax.experimental.pallas import tpu as pltpu

N_DEV = 16
M_BLK = 256
NSUB = 2
N_LANES = 2 * NSUB

RING = (0, 1, 5, 9, 13, 14, 10, 6, 2, 3, 7, 11, 15, 12, 8, 4)
POS = tuple(RING.index(l) for l in range(N_DEV))


def _lut(table, idx):
    out = jnp.int32(table[0])
    for q in range(1, N_DEV):
        out = jnp.where(idx == q, jnp.int32(table[q]), out)
    return out


class _Lane:

    def __init__(self, is_a, col0, col1, send, recv, send_sems, recv_sems):
        self.is_a = is_a
        self.col0 = col0
        self.col1 = col1
        self.send = send
        self.recv = recv
        self.send_sems = send_sems
        self.recv_sems = recv_sems
        self.rdmas = []


def kernel(x, w_mat):
    m, k = x.shape
    _, n = w_mat.shape
    ncol = n // N_LANES

    def body(x_ref, w_ref, out_ref, w_bf, *scratch):
        sends = scratch[:N_LANES]
        recvs = scratch[N_LANES:2 * N_LANES]
        send_sems = scratch[2 * N_LANES:3 * N_LANES]
        recv_sems = scratch[3 * N_LANES:4 * N_LANES]

        my = lax.axis_index("i")
        pos = _lut(POS, my)
        right = _lut(tuple(RING[(POS[l] + 1) % N_DEV] for l in range(N_DEV)),
                     my)
        left = _lut(tuple(RING[(POS[l] - 1) % N_DEV] for l in range(N_DEV)),
                    my)

        barrier = pltpu.get_barrier_semaphore()
        for nbr in (left, right):
            pl.semaphore_signal(
                barrier, inc=1, device_id=(nbr,),
                device_id_type=pl.DeviceIdType.MESH,
            )
        w_bf[...] = w_ref[...].astype(jnp.bfloat16)
        pl.semaphore_wait(barrier, 2)

        lanes = []
        for s in range(NSUB):
            lanes.append(_Lane(True, s * ncol, (s + 1) * ncol,
                               sends[2 * s], recvs[2 * s],
                               send_sems[2 * s], recv_sems[2 * s]))
            lanes.append(_Lane(False, (NSUB + s) * ncol, (NSUB + s + 1) * ncol,
                               sends[2 * s + 1], recvs[2 * s + 1],
                               send_sems[2 * s + 1], recv_sems[2 * s + 1]))

        def gemm(c, lane):
            a = x_ref[pl.ds(c * M_BLK, M_BLK), :].astype(jnp.bfloat16)
            return jnp.dot(a, w_bf[:, lane.col0:lane.col1],
                           preferred_element_type=jnp.float32)

        for t in range(N_DEV - 1):
            ca = _lut(tuple(RING[(q - 1 - t) % N_DEV] for q in range(N_DEV)),
                      pos)
            cb = _lut(tuple(RING[(q + 1 + t) % N_DEV] for q in range(N_DEV)),
                      pos)
            for lane in lanes:
                acc = gemm(ca if lane.is_a else cb, lane)
                if t > 0:
                    lane.rdmas[t - 1].wait_recv()
                    acc = acc + lane.recv[t - 1].astype(jnp.float32)
                if t >= 2:
                    lane.rdmas[t - 2].wait_send()
                lane.send[t % 2] = acc.astype(jnp.bfloat16)
                rdma = pltpu.make_async_remote_copy(
                    src_ref=lane.send.at[t % 2],
                    dst_ref=lane.recv.at[t],
                    send_sem=lane.send_sems.at[t % 2],
                    recv_sem=lane.recv_sems.at[t],
                    device_id=(right if lane.is_a else left,),
                    device_id_type=pl.DeviceIdType.MESH,
                )
                rdma.start()
                lane.rdmas.append(rdma)

        owns = [gemm(my, lane) for lane in lanes]
        for lane, own in zip(lanes, owns):
            lane.rdmas[N_DEV - 2].wait_recv()
            out_ref[:, lane.col0:lane.col1] = jnp.maximum(
                own + lane.recv[N_DEV - 2].astype(jnp.float32), 0.0)

        for lane in lanes:
            lane.rdmas[N_DEV - 3].wait_send()
            lane.rdmas[N_DEV - 2].wait_send()

    return pl.pallas_call(
        body,
        out_shape=jax.ShapeDtypeStruct((M_BLK, n), jnp.float32),
        in_specs=[
            pl.BlockSpec(memory_space=pltpu.VMEM),
            pl.BlockSpec(memory_space=pltpu.VMEM),
        ],
        out_specs=pl.BlockSpec(memory_space=pltpu.VMEM),
        scratch_shapes=(
            [pltpu.VMEM((k, n), jnp.bfloat16)]
            + [pltpu.VMEM((2, M_BLK, ncol), jnp.bfloat16)] * N_LANES
            + [pltpu.VMEM((N_DEV - 1, M_BLK, ncol), jnp.bfloat16)] * N_LANES
            + [pltpu.SemaphoreType.DMA((2,))] * N_LANES
            + [pltpu.SemaphoreType.DMA((N_DEV - 1,))] * N_LANES
        ),
        compiler_params=pltpu.CompilerParams(collective_id=0),
    )(x, w_mat)
